# speedup vs baseline: 1.2483x; 1.2483x over previous
import functools

import jax
import jax.numpy as jnp
from jax import lax
from jax.experimental import pallas as pl
from jax.experimental.pallas import tpu as pltpu, tpu_sc as plsc

NUM_ACTIONS = 100000
D = 128
B = 16384
HALF = B // 2
NC, NS = 2, 16
NW = NC * NS
ROWS_PER_W = HALF // NW
CHUNK = 128
NCHUNK = ROWS_PER_W // CHUNK

_mesh = plsc.VectorSubcoreMesh(core_axis_name="c", subcore_axis_name="s")


@functools.partial(
    pl.kernel,
    out_type=jax.ShapeDtypeStruct((B, D), jnp.float32),
    mesh=_mesh,
    scratch_types=[
        pltpu.VMEM((CHUNK,), jnp.int32),
        pltpu.VMEM((CHUNK,), jnp.int32),
        pltpu.VMEM((CHUNK, D), jnp.float32),
        pltpu.VMEM((CHUNK, D), jnp.float32),
        pltpu.VMEM((ROWS_PER_W, D), jnp.float32),
        pltpu.SemaphoreType.DMA,
        pltpu.SemaphoreType.DMA,
    ],
)
def _embed_gather(idx_hbm, table_hbm, zeros_hbm, out_hbm,
                  idx0, idx1, rows0, rows1, zbuf, sem0, sem1):
    wid = lax.axis_index("s") * NC + lax.axis_index("c")
    base = wid * ROWS_PER_W

    idxb = (idx0, idx1)
    rowsb = (rows0, rows1)
    sems = (sem0, sem1)

    copies = []
    for j in range(NCHUNK):
        pltpu.sync_copy(idx_hbm.at[pl.ds(base + j * CHUNK, CHUNK)], idxb[j])
        copies.append(pltpu.async_copy(table_hbm.at[idxb[j]], rowsb[j], sems[j]))

    pltpu.sync_copy(zeros_hbm, zbuf)
    pltpu.sync_copy(zbuf, out_hbm.at[pl.ds(base, ROWS_PER_W)])

    for j in range(NCHUNK):
        copies[j].wait()
        pltpu.sync_copy(
            rowsb[j], out_hbm.at[pl.ds(HALF + base + j * CHUNK, CHUNK)])


def kernel(input, action_embedding):
    idx_cond = input.reshape(B)[HALF:].astype(jnp.int32)
    zeros = jnp.zeros((ROWS_PER_W, D), jnp.float32)
    out = _embed_gather(idx_cond, action_embedding, zeros)
    return out[None]


# speedup vs baseline: 1.5462x; 1.2387x over previous
import functools

import jax
import jax.numpy as jnp
from jax import lax
from jax.experimental import pallas as pl
from jax.experimental.pallas import tpu as pltpu, tpu_sc as plsc

NUM_ACTIONS = 100000
D = 128
B = 16384
HALF = B // 2
NC, NS = 2, 16
NW = NC * NS
ROWS_PER_W = HALF // NW
CHUNK = 128
NCHUNK = ROWS_PER_W // CHUNK
ZROWS = 64

_mesh = plsc.VectorSubcoreMesh(core_axis_name="c", subcore_axis_name="s")


@functools.partial(
    pl.kernel,
    out_type=jax.ShapeDtypeStruct((B, D), jnp.float32),
    mesh=_mesh,
    scratch_types=[
        pltpu.VMEM((CHUNK,), jnp.int32),
        pltpu.VMEM((CHUNK,), jnp.int32),
        pltpu.VMEM((CHUNK, D), jnp.float32),
        pltpu.VMEM((CHUNK, D), jnp.float32),
        pltpu.VMEM((ZROWS, D), jnp.float32),
        pltpu.SemaphoreType.DMA,
        pltpu.SemaphoreType.DMA,
    ],
)
def _embed_gather(idx_hbm, table_hbm, out_hbm,
                  idx0, idx1, rows0, rows1, zbuf, sem0, sem1):
    wid = lax.axis_index("s") * NC + lax.axis_index("c")
    base = wid * ROWS_PER_W

    idxb = (idx0, idx1)
    rowsb = (rows0, rows1)
    sems = (sem0, sem1)

    copies = []
    for j in range(NCHUNK):
        pltpu.sync_copy(idx_hbm.at[pl.ds(base + j * CHUNK, CHUNK)], idxb[j])
        copies.append(pltpu.async_copy(table_hbm.at[idxb[j]], rowsb[j], sems[j]))

    z16 = jnp.zeros((16,), jnp.float32)

    def _zfill(i, carry):
        for k in range(D // 16):
            zbuf[i, pl.ds(k * 16, 16)] = z16
        return carry

    lax.fori_loop(0, ZROWS, _zfill, 0)
    for z in range(ROWS_PER_W // ZROWS):
        pltpu.sync_copy(zbuf, out_hbm.at[pl.ds(base + z * ZROWS, ZROWS)])

    for j in range(NCHUNK):
        copies[j].wait()
        pltpu.sync_copy(
            rowsb[j], out_hbm.at[pl.ds(HALF + base + j * CHUNK, CHUNK)])


def kernel(input, action_embedding):
    idx_cond = input.reshape(B)[HALF:].astype(jnp.int32)
    out = _embed_gather(idx_cond, action_embedding)
    return out[None]
